# Initial kernel scaffold; baseline (speedup 1.0000x reference)
import functools

import jax
import jax.numpy as jnp
from jax import lax
from jax.experimental import pallas as pl
from jax.experimental.pallas import tpu as pltpu
from jax.experimental.pallas import tpu_sc as plsc

TOKEN_VOCAB = 100000
CHAR_VOCAB = 1000
D = 128
B, L = 4096, 50
BL = B * L

NC, NS = 2, 16
NW = NC * NS
PER_W = BL // NW
CHUNK = 256
NCHUNK = PER_W // CHUNK


def _tt_body(x_ref, w_ref, o_ref):
    o_ref[...] = lax.dot_general(
        x_ref[...], w_ref[...],
        dimension_numbers=(((1,), (1,)), ((), ())),
        preferred_element_type=jnp.float32,
    )


def _ct_body(x_ref, w_ref, b_ref, o_ref):
    o_ref[...] = lax.dot_general(
        x_ref[...], w_ref[...],
        dimension_numbers=(((1,), (1,)), ((), ())),
        preferred_element_type=jnp.float32,
    ) + b_ref[...]


def _transform_token_table(token_table, w1):
    blk = 10000
    grid = TOKEN_VOCAB // blk
    return pl.pallas_call(
        _tt_body,
        grid=(grid,),
        in_specs=[
            pl.BlockSpec((blk, D), lambda i: (i, 0)),
            pl.BlockSpec((D, D), lambda i: (0, 0)),
        ],
        out_specs=pl.BlockSpec((blk, D), lambda i: (i, 0)),
        out_shape=jax.ShapeDtypeStruct((TOKEN_VOCAB, D), jnp.float32),
    )(token_table, w1)


def _transform_char_table(char_table, w2, b):
    return pl.pallas_call(
        _ct_body,
        out_shape=jax.ShapeDtypeStruct((CHAR_VOCAB, D), jnp.float32),
    )(char_table, w2, b.reshape(1, D))


def _sc_gather_body(tt_hbm, ct_hbm, tok_hbm, chr_hbm, out_hbm,
                    tok_v, chr_v, rows_v, sem):
    wid = lax.axis_index("s") * NC + lax.axis_index("c")
    base = wid * PER_W
    pltpu.sync_copy(tok_hbm.at[pl.ds(base, PER_W)], tok_v)
    pltpu.sync_copy(chr_hbm.at[pl.ds(base, PER_W)], chr_v)

    def chunk(k, carry):
        off = pl.multiple_of(k * CHUNK, CHUNK)
        pltpu.async_copy(tt_hbm.at[tok_v.at[pl.ds(off, CHUNK)]], rows_v, sem).wait()
        pltpu.async_copy(ct_hbm.at[chr_v.at[pl.ds(off, CHUNK)]], rows_v, sem,
                         add=True).wait()
        pltpu.sync_copy(rows_v, out_hbm.at[pl.ds(base + off, CHUNK)])
        return carry

    lax.fori_loop(0, NCHUNK, chunk, 0)


_sc_gather = functools.partial(
    pl.kernel,
    out_type=jax.ShapeDtypeStruct((BL, D), jnp.float32),
    mesh=plsc.VectorSubcoreMesh(core_axis_name="c", subcore_axis_name="s"),
    scratch_types=[
        pltpu.VMEM((PER_W,), jnp.int32),
        pltpu.VMEM((PER_W,), jnp.int32),
        pltpu.VMEM((CHUNK, D), jnp.float32),
        pltpu.SemaphoreType.DMA,
    ],
)(_sc_gather_body)


def kernel(tokens, chars, token_table, char_table, W, b):
    w1 = W[:, :D]
    w2 = W[:, D:]
    tt = _transform_token_table(token_table, w1)
    ct = _transform_char_table(char_table, w2, b)
    out = _sc_gather(tt, ct, tokens.reshape(-1), chars.reshape(-1))
    return out.reshape(B, L, D)


# speedup vs baseline: 4.0164x; 4.0164x over previous
import functools

import jax
import jax.numpy as jnp
from jax import lax
from jax.experimental import pallas as pl
from jax.experimental.pallas import tpu as pltpu
from jax.experimental.pallas import tpu_sc as plsc

TOKEN_VOCAB = 100000
CHAR_VOCAB = 1000
D = 128
B, L = 4096, 50
BL = B * L

NC, NS = 2, 16
NW = NC * NS
PER_W = BL // NW
CHUNK = 256
NCHUNK = PER_W // CHUNK


def _tt_body(x_ref, w_ref, o_ref):
    o_ref[...] = lax.dot_general(
        x_ref[...], w_ref[...],
        dimension_numbers=(((1,), (1,)), ((), ())),
        preferred_element_type=jnp.float32,
    )


def _ct_body(x_ref, w_ref, b_ref, o_ref):
    o_ref[...] = lax.dot_general(
        x_ref[...], w_ref[...],
        dimension_numbers=(((1,), (1,)), ((), ())),
        preferred_element_type=jnp.float32,
    ) + b_ref[...]


def _transform_token_table(token_table, w1):
    blk = 10000
    grid = TOKEN_VOCAB // blk
    return pl.pallas_call(
        _tt_body,
        grid=(grid,),
        in_specs=[
            pl.BlockSpec((blk, D), lambda i: (i, 0)),
            pl.BlockSpec((D, D), lambda i: (0, 0)),
        ],
        out_specs=pl.BlockSpec((blk, D), lambda i: (i, 0)),
        out_shape=jax.ShapeDtypeStruct((TOKEN_VOCAB, D), jnp.float32),
    )(token_table, w1)


def _transform_char_table(char_table, w2, b):
    return pl.pallas_call(
        _ct_body,
        out_shape=jax.ShapeDtypeStruct((CHAR_VOCAB, D), jnp.float32),
    )(char_table, w2, b.reshape(1, D))


def _sc_gather_body(tt_hbm, ct_hbm, tok_hbm, chr_hbm, out_hbm,
                    tok_v, chr_v, rows_v, sem):
    wid = lax.axis_index("s") * NC + lax.axis_index("c")
    base = wid * PER_W
    pltpu.sync_copy(tok_hbm.at[pl.ds(base, PER_W)], tok_v)
    pltpu.sync_copy(chr_hbm.at[pl.ds(base, PER_W)], chr_v)

    def chunk(k, carry):
        off = pl.multiple_of(k * CHUNK, CHUNK)
        pltpu.async_copy(tt_hbm.at[tok_v.at[pl.ds(off, CHUNK)]], rows_v, sem).wait()
        pltpu.async_copy(ct_hbm.at[chr_v.at[pl.ds(off, CHUNK)]], rows_v, sem,
                         add=True).wait()
        pltpu.sync_copy(rows_v, out_hbm.at[pl.ds(base + off, CHUNK)])
        return carry

    lax.fori_loop(0, NCHUNK, chunk, 0)


@functools.cache
def _sc_gather():
    return pl.kernel(
        _sc_gather_body,
        out_type=jax.ShapeDtypeStruct((BL, D), jnp.float32),
        mesh=plsc.VectorSubcoreMesh(core_axis_name="c", subcore_axis_name="s"),
        scratch_types=[
            pltpu.VMEM((PER_W,), jnp.int32),
            pltpu.VMEM((PER_W,), jnp.int32),
            pltpu.VMEM((CHUNK, D), jnp.float32),
            pltpu.SemaphoreType.DMA,
        ],
    )


def kernel(tokens, chars, token_table, char_table, W, b):
    w1 = W[:, :D]
    w2 = W[:, D:]
    tt = _transform_token_table(token_table, w1)
    ct = _transform_char_table(char_table, w2, b)
    out = _sc_gather()(tt, ct, tokens.reshape(-1), chars.reshape(-1))
    return out.reshape(B, L, D)


# speedup vs baseline: 12.6910x; 3.1598x over previous
import functools

import jax
import jax.numpy as jnp
from jax import lax
from jax.experimental import pallas as pl
from jax.experimental.pallas import tpu as pltpu
from jax.experimental.pallas import tpu_sc as plsc

TOKEN_VOCAB = 100000
CHAR_VOCAB = 1000
D = 128
B, L = 4096, 50
BL = B * L

NC, NS = 2, 16
NW = NC * NS
PER_W = BL // NW
NBUF = 5
CHUNK = 160
NCHUNK = PER_W // CHUNK
NGROUP = NCHUNK // NBUF


def _transform_body(tok_tab_ref, chr_tab_ref, w_ref, b_ref, tt_ref, ct_ref):
    w1 = w_ref[:, :D]
    tt_ref[...] = lax.dot_general(
        tok_tab_ref[...], w1,
        dimension_numbers=(((1,), (1,)), ((), ())),
        preferred_element_type=jnp.float32,
    )

    @pl.when(pl.program_id(0) == 0)
    def _():
        w2 = w_ref[:, D:]
        ct_ref[...] = lax.dot_general(
            chr_tab_ref[...], w2,
            dimension_numbers=(((1,), (1,)), ((), ())),
            preferred_element_type=jnp.float32,
        ) + b_ref[...]


def _transform_tables(token_table, char_table, W, b):
    blk = 10000
    grid = TOKEN_VOCAB // blk
    return pl.pallas_call(
        _transform_body,
        grid=(grid,),
        in_specs=[
            pl.BlockSpec((blk, D), lambda i: (i, 0)),
            pl.BlockSpec((CHAR_VOCAB, D), lambda i: (0, 0)),
            pl.BlockSpec((D, 2 * D), lambda i: (0, 0)),
            pl.BlockSpec((1, D), lambda i: (0, 0)),
        ],
        out_specs=[
            pl.BlockSpec((blk, D), lambda i: (i, 0)),
            pl.BlockSpec((CHAR_VOCAB, D), lambda i: (0, 0)),
        ],
        out_shape=[
            jax.ShapeDtypeStruct((TOKEN_VOCAB, D), jnp.float32),
            jax.ShapeDtypeStruct((CHAR_VOCAB, D), jnp.float32),
        ],
    )(token_table, char_table, W, b.reshape(1, D))


def _sc_gather_body(tt_hbm, ct_hbm, tok_hbm, chr_hbm, out_hbm,
                    tok_v, chr_v, ct_sh, *scratch):
    rows = scratch[:NBUF]
    tsem = scratch[NBUF:2 * NBUF]
    csem = scratch[2 * NBUF:3 * NBUF]
    osem = scratch[3 * NBUF:4 * NBUF]

    wid = lax.axis_index("s") * NC + lax.axis_index("c")
    base = wid * PER_W

    @pl.when(lax.axis_index("s") == 0)
    def _():
        pltpu.sync_copy(ct_hbm, ct_sh)

    pltpu.sync_copy(tok_hbm.at[pl.ds(base, PER_W)], tok_v)
    pltpu.sync_copy(chr_hbm.at[pl.ds(base, PER_W)], chr_v)
    plsc.subcore_barrier()

    def tt_gather(k, buf, sem):
        off = pl.multiple_of(k * CHUNK, CHUNK)
        pltpu.async_copy(tt_hbm.at[tok_v.at[pl.ds(off, CHUNK)]], buf, sem)

    def ct_add(k, buf, sem):
        off = pl.multiple_of(k * CHUNK, CHUNK)
        pltpu.async_copy(ct_sh.at[chr_v.at[pl.ds(off, CHUNK)]], buf, sem,
                         add=True)

    def scatter(k, buf, sem):
        off = pl.multiple_of(k * CHUNK, CHUNK)
        pltpu.async_copy(buf, out_hbm.at[pl.ds(base + off, CHUNK)], sem)

    def wait_gather(table, idx_v, buf, sem):
        pltpu.make_async_copy(table.at[idx_v.at[pl.ds(0, CHUNK)]], buf, sem).wait()

    def wait_scatter(buf, sem):
        pltpu.make_async_copy(buf, out_hbm.at[pl.ds(base, CHUNK)], sem).wait()

    for b in range(NBUF - 2):
        tt_gather(b, rows[b], tsem[b])

    def group(g, carry):
        for b in range(NBUF):
            k = g * NBUF + b
            bp = (b - 2) % NBUF
            wait_gather(tt_hbm, tok_v, rows[b], tsem[b])
            ct_add(k, rows[b], csem[b])
            wait_gather(ct_sh, chr_v, rows[b], csem[b])
            scatter(k, rows[b], osem[b])
            j = k + NBUF - 2

            @pl.when(jnp.logical_and(j < NCHUNK, k >= 2))
            def _():
                wait_scatter(rows[bp], osem[bp])

            @pl.when(j < NCHUNK)
            def _():
                tt_gather(j, rows[bp], tsem[bp])
        return carry

    lax.fori_loop(0, NGROUP, group, 0)
    for b in range(NBUF):
        wait_scatter(rows[b], osem[b])


@functools.cache
def _sc_gather():
    return pl.kernel(
        _sc_gather_body,
        out_type=jax.ShapeDtypeStruct((BL, D), jnp.float32),
        mesh=plsc.VectorSubcoreMesh(core_axis_name="c", subcore_axis_name="s"),
        scratch_types=[
            pltpu.VMEM((PER_W,), jnp.int32),
            pltpu.VMEM((PER_W,), jnp.int32),
            pltpu.VMEM_SHARED((CHAR_VOCAB, D), jnp.float32),
        ]
        + [pltpu.VMEM((CHUNK, D), jnp.float32)] * NBUF
        + [pltpu.SemaphoreType.DMA] * (3 * NBUF),
    )


def kernel(tokens, chars, token_table, char_table, W, b):
    tt, ct = _transform_tables(token_table, char_table, W, b)
    out = _sc_gather()(tt, ct, tokens.T.reshape(-1), chars.T.reshape(-1))
    return out.reshape(L, B, D).transpose(1, 0, 2)
